# baseline (speedup 1.0000x reference)
import functools

import jax
import jax.numpy as jnp
from jax import lax
from jax.experimental import pallas as pl
from jax.experimental.pallas import tpu as pltpu
from jax.experimental.pallas import tpu_sc as plsc

_LN_EPS = 1e-12
_NUM_WORKERS = 32
_GATHER_CHUNK = 32


def _sc_gather(table, ids2d, seg0, seg):
    h = table.shape[1]
    n = ids2d.shape[0] * seg
    b_per_w = n // _NUM_WORKERS
    wpr = seg // b_per_w
    nchunk = b_per_w // _GATHER_CHUNK
    mesh = plsc.VectorSubcoreMesh(core_axis_name="c", subcore_axis_name="s")

    @functools.partial(
        pl.kernel,
        out_type=jax.ShapeDtypeStruct((n, h), table.dtype),
        mesh=mesh,
        scratch_types=(
            [pltpu.VMEM((b_per_w,), jnp.int32)]
            + [pltpu.VMEM((_GATHER_CHUNK, h), table.dtype)
               for _ in range(nchunk)]
            + [pltpu.SemaphoreType.DMA, pltpu.SemaphoreType.DMA]
        ),
    )
    def k(tab_hbm, idx_hbm, out_hbm, idx_v, *rest):
        bufs, (gsem, wsem) = rest[:nchunk], rest[nchunk:]
        wid = lax.axis_index("s") * 2 + lax.axis_index("c")
        base = wid * b_per_w
        brow = wid // wpr
        scol = seg0 + (wid % wpr) * b_per_w
        pltpu.sync_copy(idx_hbm.at[brow, pl.ds(scol, b_per_w)], idx_v)

        gpend = [
            pltpu.async_copy(
                tab_hbm.at[idx_v.at[pl.ds(j * _GATHER_CHUNK, _GATHER_CHUNK)]],
                bufs[j], gsem)
            for j in range(nchunk)
        ]
        wpend = []
        for j in range(nchunk):
            gpend[j].wait()
            wpend.append(pltpu.async_copy(
                bufs[j],
                out_hbm.at[pl.ds(base + j * _GATHER_CHUNK, _GATHER_CHUNK)],
                wsem))
        for w in wpend:
            w.wait()

    return k(table, ids2d)


def _ln_math(g_ref, p_ref, t_ref, w_ref, o_ref):
    tt = t_ref[0, 0, :].astype(jnp.float32)[:, None]
    base = w_ref[0:1, :]
    diff = w_ref[1:2, :] - w_ref[0:1, :]
    x = g_ref[...] + p_ref[...] + base + tt * diff
    mean = jnp.mean(x, axis=1, keepdims=True)
    xc = x - mean
    var = jnp.mean(xc * xc, axis=1, keepdims=True)
    y = xc * lax.rsqrt(var + _LN_EPS)
    o_ref[...] = y * w_ref[2:3, :] + w_ref[3:4, :]


def _tc_ln_body(g_ref, p_ref, t_ref, w_ref, o_ref):
    _ln_math(g_ref, p_ref, t_ref, w_ref, o_ref)


def _tc_ln_body_alias(d_ref, g_ref, p_ref, t_ref, w_ref, o_ref):
    del d_ref
    _ln_math(g_ref, p_ref, t_ref, w_ref, o_ref)


def _tc_ln(dst, n_total, c_idx, gathered, pos_emb, tt3, wvec, block):
    nc, h = gathered.shape
    s = pos_emb.shape[0]
    b = tt3.shape[0]
    seg = nc // b
    sbg = seg // block
    sb = s // block
    c0 = c_idx * sbg
    data_specs = [
        pl.BlockSpec((block, h), lambda j, i: (i * sbg + j, 0)),
        pl.BlockSpec((block, h), lambda j, i: (c0 + j, 0)),
        pl.BlockSpec((1, 1, block), lambda j, i: (i, 0, c0 + j)),
        pl.BlockSpec((4, h), lambda j, i: (0, 0)),
    ]
    out_spec = pl.BlockSpec((block, h), lambda j, i: (i * sb + c0 + j, 0))
    common = dict(
        grid=(sbg, b),
        out_specs=out_spec,
        out_shape=jax.ShapeDtypeStruct((n_total, h), jnp.float32),
        compiler_params=pltpu.CompilerParams(
            dimension_semantics=("parallel", "parallel"),
        ),
    )
    args = (gathered, pos_emb, tt3, wvec)
    if dst is None:
        return pl.pallas_call(_tc_ln_body, in_specs=data_specs, **common)(*args)
    return pl.pallas_call(
        _tc_ln_body_alias,
        in_specs=[pl.BlockSpec(memory_space=pl.ANY)] + data_specs,
        input_output_aliases={0: 0},
        **common,
    )(dst, *args)


_OVERLAP_CHUNKS = 4
_TC_BLOCK = 512


def kernel(input_ids, token_type_ids, word_embeddings, position_embeddings,
           token_type_embeddings, ln_gamma, ln_beta):
    b, s = input_ids.shape
    h = word_embeddings.shape[1]
    ids2d = input_ids.astype(jnp.int32)
    tt3 = token_type_ids.astype(jnp.int32).reshape(b, 1, s)
    wvec = jnp.concatenate(
        [token_type_embeddings, ln_gamma.reshape(1, h),
         ln_beta.reshape(1, h)], axis=0)
    c = _OVERLAP_CHUNKS if s % _OVERLAP_CHUNKS == 0 else 1
    seg = s // c
    dst = None
    for i in range(c):
        g_i = _sc_gather(word_embeddings, ids2d, i * seg, seg)
        dst = _tc_ln(dst, b * s, i, g_i,
                     position_embeddings[:s], tt3, wvec, block=_TC_BLOCK)
    return dst.reshape(b, s, h)


# speedup vs baseline: 1.0828x; 1.0828x over previous
import functools

import jax
import jax.numpy as jnp
from jax import lax
from jax.experimental import pallas as pl
from jax.experimental.pallas import tpu as pltpu
from jax.experimental.pallas import tpu_sc as plsc

_LN_EPS = 1e-12
_NUM_WORKERS = 32
_GATHER_CHUNK = 32


def _sc_gather(table, ids2d, seg0, seg):
    h = table.shape[1]
    n = ids2d.shape[0] * seg
    b_per_w = n // _NUM_WORKERS
    wpr = seg // b_per_w
    nchunk = b_per_w // _GATHER_CHUNK
    mesh = plsc.VectorSubcoreMesh(core_axis_name="c", subcore_axis_name="s")

    @functools.partial(
        pl.kernel,
        out_type=jax.ShapeDtypeStruct((n, h), table.dtype),
        mesh=mesh,
        scratch_types=(
            [pltpu.VMEM((b_per_w,), jnp.int32)]
            + [pltpu.VMEM((_GATHER_CHUNK, h), table.dtype)
               for _ in range(nchunk)]
            + [pltpu.SemaphoreType.DMA, pltpu.SemaphoreType.DMA]
        ),
    )
    def k(tab_hbm, idx_hbm, out_hbm, idx_v, *rest):
        bufs, (gsem, wsem) = rest[:nchunk], rest[nchunk:]
        wid = lax.axis_index("s") * 2 + lax.axis_index("c")
        base = wid * b_per_w
        brow = wid // wpr
        scol = seg0 + (wid % wpr) * b_per_w
        pltpu.sync_copy(idx_hbm.at[brow, pl.ds(scol, b_per_w)], idx_v)

        gpend = [
            pltpu.async_copy(
                tab_hbm.at[idx_v.at[pl.ds(j * _GATHER_CHUNK, _GATHER_CHUNK)]],
                bufs[j], gsem)
            for j in range(nchunk)
        ]
        wpend = []
        for j in range(nchunk):
            gpend[j].wait()
            wpend.append(pltpu.async_copy(
                bufs[j],
                out_hbm.at[pl.ds(base + j * _GATHER_CHUNK, _GATHER_CHUNK)],
                wsem))
        for w in wpend:
            w.wait()

    return k(table, ids2d)


def _ln_math(g_ref, p_ref, t_ref, w_ref, o_ref):
    tt = t_ref[0, 0, :].astype(jnp.float32)[:, None]
    base = w_ref[0:1, :]
    diff = w_ref[1:2, :] - w_ref[0:1, :]
    x = g_ref[...] + p_ref[...] + base + tt * diff
    mean = jnp.mean(x, axis=1, keepdims=True)
    xc = x - mean
    var = jnp.mean(xc * xc, axis=1, keepdims=True)
    y = xc * lax.rsqrt(var + _LN_EPS)
    o_ref[...] = y * w_ref[2:3, :] + w_ref[3:4, :]


def _tc_ln_body(g_ref, p_ref, t_ref, w_ref, o_ref):
    _ln_math(g_ref, p_ref, t_ref, w_ref, o_ref)


def _tc_ln_body_alias(d_ref, g_ref, p_ref, t_ref, w_ref, o_ref):
    del d_ref
    _ln_math(g_ref, p_ref, t_ref, w_ref, o_ref)


def _tc_ln(dst, n_total, c_idx, gathered, pos_emb, tt3, wvec, block):
    nc, h = gathered.shape
    s = pos_emb.shape[0]
    b = tt3.shape[0]
    seg = nc // b
    sbg = seg // block
    sb = s // block
    c0 = c_idx * sbg
    data_specs = [
        pl.BlockSpec((block, h), lambda j, i: (i * sbg + j, 0)),
        pl.BlockSpec((block, h), lambda j, i: (c0 + j, 0)),
        pl.BlockSpec((1, 1, block), lambda j, i: (i, 0, c0 + j)),
        pl.BlockSpec((4, h), lambda j, i: (0, 0)),
    ]
    out_spec = pl.BlockSpec((block, h), lambda j, i: (i * sb + c0 + j, 0))
    common = dict(
        grid=(sbg, b),
        out_specs=out_spec,
        out_shape=jax.ShapeDtypeStruct((n_total, h), jnp.float32),
        compiler_params=pltpu.CompilerParams(
            dimension_semantics=("parallel", "parallel"),
        ),
    )
    args = (gathered, pos_emb, tt3, wvec)
    if dst is None:
        return pl.pallas_call(_tc_ln_body, in_specs=data_specs, **common)(*args)
    return pl.pallas_call(
        _tc_ln_body_alias,
        in_specs=[pl.BlockSpec(memory_space=pl.ANY)] + data_specs,
        input_output_aliases={0: 0},
        **common,
    )(dst, *args)


_OVERLAP_CHUNKS = 2
_TC_BLOCK = 1024


def kernel(input_ids, token_type_ids, word_embeddings, position_embeddings,
           token_type_embeddings, ln_gamma, ln_beta):
    b, s = input_ids.shape
    h = word_embeddings.shape[1]
    ids2d = input_ids.astype(jnp.int32)
    tt3 = token_type_ids.astype(jnp.int32).reshape(b, 1, s)
    wvec = jnp.concatenate(
        [token_type_embeddings, ln_gamma.reshape(1, h),
         ln_beta.reshape(1, h)], axis=0)
    c = _OVERLAP_CHUNKS if s % _OVERLAP_CHUNKS == 0 else 1
    seg = s // c
    dst = None
    for i in range(c):
        g_i = _sc_gather(word_embeddings, ids2d, i * seg, seg)
        dst = _tc_ln(dst, b * s, i, g_i,
                     position_embeddings[:s], tt3, wvec, block=_TC_BLOCK)
    return dst.reshape(b, s, h)


# speedup vs baseline: 1.0895x; 1.0061x over previous
import functools

import jax
import jax.numpy as jnp
from jax import lax
from jax.experimental import pallas as pl
from jax.experimental.pallas import tpu as pltpu
from jax.experimental.pallas import tpu_sc as plsc

_LN_EPS = 1e-12
_NUM_WORKERS = 32
_GATHER_CHUNK = 64


def _sc_gather(table, ids2d, seg0, seg):
    h = table.shape[1]
    n = ids2d.shape[0] * seg
    b_per_w = n // _NUM_WORKERS
    wpr = seg // b_per_w
    nchunk = b_per_w // _GATHER_CHUNK
    mesh = plsc.VectorSubcoreMesh(core_axis_name="c", subcore_axis_name="s")

    @functools.partial(
        pl.kernel,
        out_type=jax.ShapeDtypeStruct((n, h), table.dtype),
        mesh=mesh,
        scratch_types=(
            [pltpu.VMEM((b_per_w,), jnp.int32)]
            + [pltpu.VMEM((_GATHER_CHUNK, h), table.dtype)
               for _ in range(nchunk)]
            + [pltpu.SemaphoreType.DMA, pltpu.SemaphoreType.DMA]
        ),
    )
    def k(tab_hbm, idx_hbm, out_hbm, idx_v, *rest):
        bufs, (gsem, wsem) = rest[:nchunk], rest[nchunk:]
        wid = lax.axis_index("s") * 2 + lax.axis_index("c")
        base = wid * b_per_w
        brow = wid // wpr
        scol = seg0 + (wid % wpr) * b_per_w
        pltpu.sync_copy(idx_hbm.at[brow, pl.ds(scol, b_per_w)], idx_v)

        gpend = [
            pltpu.async_copy(
                tab_hbm.at[idx_v.at[pl.ds(j * _GATHER_CHUNK, _GATHER_CHUNK)]],
                bufs[j], gsem)
            for j in range(nchunk)
        ]
        wpend = []
        for j in range(nchunk):
            gpend[j].wait()
            wpend.append(pltpu.async_copy(
                bufs[j],
                out_hbm.at[pl.ds(base + j * _GATHER_CHUNK, _GATHER_CHUNK)],
                wsem))
        for w in wpend:
            w.wait()

    return k(table, ids2d)


def _ln_math(g_ref, p_ref, t_ref, w_ref, o_ref):
    tt = t_ref[0, 0, :].astype(jnp.float32)[:, None]
    base = w_ref[0:1, :]
    diff = w_ref[1:2, :] - w_ref[0:1, :]
    x = g_ref[...] + p_ref[...] + base + tt * diff
    mean = jnp.mean(x, axis=1, keepdims=True)
    xc = x - mean
    var = jnp.mean(xc * xc, axis=1, keepdims=True)
    y = xc * lax.rsqrt(var + _LN_EPS)
    o_ref[...] = y * w_ref[2:3, :] + w_ref[3:4, :]


def _tc_ln_body(g_ref, p_ref, t_ref, w_ref, o_ref):
    _ln_math(g_ref, p_ref, t_ref, w_ref, o_ref)


def _tc_ln_body_alias(d_ref, g_ref, p_ref, t_ref, w_ref, o_ref):
    del d_ref
    _ln_math(g_ref, p_ref, t_ref, w_ref, o_ref)


def _tc_ln(dst, n_total, c_idx, gathered, pos_emb, tt3, wvec, block):
    nc, h = gathered.shape
    s = pos_emb.shape[0]
    b = tt3.shape[0]
    seg = nc // b
    sbg = seg // block
    sb = s // block
    c0 = c_idx * sbg
    data_specs = [
        pl.BlockSpec((block, h), lambda j, i: (i * sbg + j, 0)),
        pl.BlockSpec((block, h), lambda j, i: (c0 + j, 0)),
        pl.BlockSpec((1, 1, block), lambda j, i: (i, 0, c0 + j)),
        pl.BlockSpec((4, h), lambda j, i: (0, 0)),
    ]
    out_spec = pl.BlockSpec((block, h), lambda j, i: (i * sb + c0 + j, 0))
    common = dict(
        grid=(sbg, b),
        out_specs=out_spec,
        out_shape=jax.ShapeDtypeStruct((n_total, h), jnp.float32),
        compiler_params=pltpu.CompilerParams(
            dimension_semantics=("parallel", "parallel"),
        ),
    )
    args = (gathered, pos_emb, tt3, wvec)
    if dst is None:
        return pl.pallas_call(_tc_ln_body, in_specs=data_specs, **common)(*args)
    return pl.pallas_call(
        _tc_ln_body_alias,
        in_specs=[pl.BlockSpec(memory_space=pl.ANY)] + data_specs,
        input_output_aliases={0: 0},
        **common,
    )(dst, *args)


_OVERLAP_CHUNKS = 2
_TC_BLOCK = 1024


def kernel(input_ids, token_type_ids, word_embeddings, position_embeddings,
           token_type_embeddings, ln_gamma, ln_beta):
    b, s = input_ids.shape
    h = word_embeddings.shape[1]
    ids2d = input_ids.astype(jnp.int32)
    tt3 = token_type_ids.astype(jnp.int32).reshape(b, 1, s)
    wvec = jnp.concatenate(
        [token_type_embeddings, ln_gamma.reshape(1, h),
         ln_beta.reshape(1, h)], axis=0)
    c = _OVERLAP_CHUNKS if s % _OVERLAP_CHUNKS == 0 else 1
    seg = s // c
    dst = None
    for i in range(c):
        g_i = _sc_gather(word_embeddings, ids2d, i * seg, seg)
        dst = _tc_ln(dst, b * s, i, g_i,
                     position_embeddings[:s], tt3, wvec, block=_TC_BLOCK)
    return dst.reshape(b, s, h)
